# Initial kernel scaffold; baseline (speedup 1.0000x reference)
import jax
import jax.numpy as jnp
from jax.experimental import pallas as pl


def kernel(input_features, node_neigh_index, prob_retained, W1, b1, g1, bt1, W2, b2, g2, bt2):
    raise NotImplementedError("write your pallas kernel here")


# speedup vs baseline: 1.4406x; 1.4406x over previous
import functools

import jax
import jax.numpy as jnp
from jax import lax
from jax.experimental import pallas as pl
from jax.experimental.pallas import tpu as pltpu
from jax.experimental.pallas import tpu_sc as plsc

N, K, C = 10000, 32, 128
_EPS = 1e-5


def _mlp_body(x_ref, w1t_ref, b1_ref, g1_ref, bt1_ref, w2t_ref, b2_ref,
              g2_ref, bt2_ref, f_ref):
    x = x_ref[...]

    h = jnp.dot(x, w1t_ref[...], preferred_element_type=jnp.float32)
    h = h + b1_ref[...]
    h = jnp.where(h > 0, h, 0.2 * h)
    mu = jnp.mean(h, axis=0, keepdims=True)
    var = jnp.mean(jnp.square(h - mu), axis=0, keepdims=True)
    h = (h - mu) * (g1_ref[...] * lax.rsqrt(var + _EPS)) + bt1_ref[...]

    h = jnp.dot(h, w2t_ref[...], preferred_element_type=jnp.float32)
    h = h + b2_ref[...]
    h = jnp.where(h > 0, h, 0.2 * h)
    mu = jnp.mean(h, axis=0, keepdims=True)
    var = jnp.mean(jnp.square(h - mu), axis=0, keepdims=True)
    h = (h - mu) * (g2_ref[...] * lax.rsqrt(var + _EPS)) + bt2_ref[...]

    f_ref[...] = h


def _mlp(x, w1t, b1, g1, bt1, w2t, b2, g2, bt2):
    return pl.pallas_call(
        _mlp_body,
        out_shape=jax.ShapeDtypeStruct((N, C), jnp.float32),
    )(x, w1t, b1, g1, bt1, w2t, b2, g2, bt2)


_NC = 2
_NS = 16
_NW = _NC * _NS
_G = 16
_CHUNKS = 20
_PW = _G * _CHUNKS
_NPAD = _NW * _PW
_VPC = C // 16


def _gather_mean_body(f_hbm, idx_hbm, out_hbm, idx_v, rows_v, out_v, sem):
    wid = lax.axis_index("s") * _NC + lax.axis_index("c")
    node_base = wid * _PW

    def chunk(ci, carry):
        nbase = node_base + ci * _G
        pltpu.sync_copy(idx_hbm.at[pl.ds(nbase * K, _G * K)], idx_v)
        pltpu.async_copy(f_hbm.at[idx_v], rows_v, sem).wait()

        for g in range(_G):
            row0 = g * K

            def kbody(k, accs):
                return tuple(
                    accs[c] + rows_v[row0 + k, pl.ds(c * 16, 16)]
                    for c in range(_VPC)
                )

            accs = lax.fori_loop(
                1, K, kbody,
                tuple(rows_v[row0, pl.ds(c * 16, 16)] for c in range(_VPC)),
            )
            for c in range(_VPC):
                out_v[g, pl.ds(c * 16, 16)] = accs[c] * (1.0 / K)

        pltpu.sync_copy(out_v, out_hbm.at[pl.ds(nbase, _G)])
        return carry

    lax.fori_loop(0, _CHUNKS, chunk, 0)


@functools.partial(jax.jit, donate_argnums=())
def _gather_mean(f, idx_flat):
    mesh = plsc.VectorSubcoreMesh(core_axis_name="c", subcore_axis_name="s")
    return pl.kernel(
        _gather_mean_body,
        out_type=jax.ShapeDtypeStruct((_NPAD, C), jnp.float32),
        mesh=mesh,
        scratch_types=[
            pltpu.VMEM((_G * K,), jnp.int32),
            pltpu.VMEM((_G * K, C), jnp.float32),
            pltpu.VMEM((_G, C), jnp.float32),
            pltpu.SemaphoreType.DMA,
        ],
    )(f, idx_flat)


def kernel(input_features, node_neigh_index, prob_retained, W1, b1, g1, bt1,
           W2, b2, g2, bt2):
    del prob_retained
    f = _mlp(
        input_features,
        W1.T,
        b1.reshape(1, C), g1.reshape(1, C), bt1.reshape(1, C),
        W2.T,
        b2.reshape(1, C), g2.reshape(1, C), bt2.reshape(1, C),
    )
    idx = node_neigh_index.astype(jnp.int32)
    idx_flat = jnp.pad(idx, ((0, _NPAD - N), (0, 0))).reshape(-1)
    node_update = _gather_mean(f, idx_flat)[:N]
    return (node_update, f)


# speedup vs baseline: 2.8386x; 1.9704x over previous
import functools

import jax
import jax.numpy as jnp
from jax import lax
from jax.experimental import pallas as pl
from jax.experimental.pallas import tpu as pltpu
from jax.experimental.pallas import tpu_sc as plsc

N, K, C = 10000, 32, 128
_EPS = 1e-5
_CP = C // 2


def _mlp_body(x_ref, w1t_ref, b1_ref, g1_ref, bt1_ref, w2t_ref, b2_ref,
              g2_ref, bt2_ref, pa_ref, pb_ref, f_ref, fp_ref):
    x = x_ref[...]

    h = jnp.dot(x, w1t_ref[...], preferred_element_type=jnp.float32)
    h = h + b1_ref[...]
    h = jnp.where(h > 0, h, 0.2 * h)
    mu = jnp.mean(h, axis=0, keepdims=True)
    var = jnp.mean(jnp.square(h - mu), axis=0, keepdims=True)
    h = (h - mu) * (g1_ref[...] * lax.rsqrt(var + _EPS)) + bt1_ref[...]

    h = jnp.dot(h, w2t_ref[...], preferred_element_type=jnp.float32)
    h = h + b2_ref[...]
    h = jnp.where(h > 0, h, 0.2 * h)
    mu = jnp.mean(h, axis=0, keepdims=True)
    var = jnp.mean(jnp.square(h - mu), axis=0, keepdims=True)
    h = (h - mu) * (g2_ref[...] * lax.rsqrt(var + _EPS)) + bt2_ref[...]

    f_ref[...] = h

    a = jnp.dot(h, pa_ref[...], preferred_element_type=jnp.float32)
    b = jnp.dot(h, pb_ref[...], preferred_element_type=jnp.float32)
    ua = lax.bitcast_convert_type(
        a.astype(jnp.bfloat16).astype(jnp.float32), jnp.uint32)
    ub = lax.bitcast_convert_type(
        b.astype(jnp.bfloat16).astype(jnp.float32), jnp.uint32)
    w = lax.bitwise_or(
        lax.shift_right_logical(ua, jnp.uint32(16)),
        lax.bitwise_and(ub, jnp.uint32(0xFFFF0000)),
    )
    fp_ref[...] = lax.bitcast_convert_type(w, jnp.int32)


def _mlp(x, w1t, b1, g1, bt1, w2t, b2, g2, bt2, pa, pb):
    return pl.pallas_call(
        _mlp_body,
        out_shape=(
            jax.ShapeDtypeStruct((N, C), jnp.float32),
            jax.ShapeDtypeStruct((N, _CP), jnp.int32),
        ),
    )(x, w1t, b1, g1, bt1, w2t, b2, g2, bt2, pa, pb)


_NC = 2
_NS = 16
_NW = _NC * _NS
_G = 16
_CHUNKS = 20
_PW = _G * _CHUNKS
_NPAD = _NW * _PW
_W16 = _CP // 16
_KU = 4


def _gather_mean_body(fp_hbm, idx_hbm, out_hbm, idx_v, rows0, rows1, out_v,
                      sem0, sem1):
    wid = lax.axis_index("s") * _NC + lax.axis_index("c")
    node_base = wid * _PW

    pltpu.sync_copy(idx_hbm.at[pl.ds(node_base * K, _PW * K)], idx_v)

    rows = (rows0, rows1)
    sems = (sem0, sem1)

    def gather_src(ci):
        return fp_hbm.at[idx_v.at[pl.ds(ci * (_G * K), _G * K)]]

    def fire(ci, b):
        pltpu.async_copy(gather_src(ci), rows[b], sems[b])

    fire(0, 0)
    fire(1, 1)

    zero = jnp.zeros((16,), jnp.float32)
    himask = jnp.full((16,), -65536, jnp.int32)

    def pair(c0, carry):
        for b in range(2):
            ci = c0 * 2 + b
            pltpu.make_async_copy(gather_src(0), rows[b], sems[b]).wait()
            rv = rows[b]
            for g in range(_G):
                row0 = g * K

                def kbody(kk, accs, rv=rv, row0=row0):
                    base = row0 + kk * _KU
                    for u in range(_KU):
                        new = list(accs)
                        for v in range(_W16):
                            w = rv[base + u, pl.ds(v * 16, 16)]
                            lo = lax.bitcast_convert_type(
                                lax.shift_left(w, 16), jnp.float32)
                            hi = lax.bitcast_convert_type(
                                lax.bitwise_and(w, himask), jnp.float32)
                            new[2 * v] = new[2 * v] + lo
                            new[2 * v + 1] = new[2 * v + 1] + hi
                        accs = tuple(new)
                    return accs

                accs = lax.fori_loop(0, K // _KU, kbody, (zero,) * (2 * _W16))
                for c in range(2 * _W16):
                    out_v[g, pl.ds(c * 16, 16)] = accs[c] * (1.0 / K)

            pltpu.sync_copy(out_v, out_hbm.at[pl.ds(node_base + ci * _G, _G)])

            @pl.when(ci + 2 < _CHUNKS)
            def _():
                fire(ci + 2, b)

        return carry

    lax.fori_loop(0, _CHUNKS // 2, pair, 0)


@functools.partial(jax.jit, donate_argnums=())
def _gather_mean(fp, idx_flat):
    mesh = plsc.VectorSubcoreMesh(core_axis_name="c", subcore_axis_name="s")
    return pl.kernel(
        _gather_mean_body,
        out_type=jax.ShapeDtypeStruct((_NPAD, C), jnp.float32),
        mesh=mesh,
        scratch_types=[
            pltpu.VMEM((_PW * K,), jnp.int32),
            pltpu.VMEM((_G * K, _CP), jnp.int32),
            pltpu.VMEM((_G * K, _CP), jnp.int32),
            pltpu.VMEM((_G, C), jnp.float32),
            pltpu.SemaphoreType.DMA,
            pltpu.SemaphoreType.DMA,
        ],
        compiler_params=pltpu.CompilerParams(use_tc_tiling_on_sc=False),
    )(fp, idx_flat)


_COLS_A = [32 * (p // 16) + p % 16 for p in range(_CP)]
_COLS_B = [32 * (p // 16) + 16 + p % 16 for p in range(_CP)]


def kernel(input_features, node_neigh_index, prob_retained, W1, b1, g1, bt1,
           W2, b2, g2, bt2):
    del prob_retained
    pa = jax.nn.one_hot(jnp.array(_COLS_A), C, axis=0, dtype=jnp.float32)
    pb = jax.nn.one_hot(jnp.array(_COLS_B), C, axis=0, dtype=jnp.float32)
    f, fp = _mlp(
        input_features,
        W1.T,
        b1.reshape(1, C), g1.reshape(1, C), bt1.reshape(1, C),
        W2.T,
        b2.reshape(1, C), g2.reshape(1, C), bt2.reshape(1, C),
        pa, pb,
    )
    idx = node_neigh_index.astype(jnp.int32)
    idx_flat = jnp.pad(idx, ((0, _NPAD - N), (0, 0))).reshape(-1)
    node_update = _gather_mean(fp, idx_flat)[:N]
    return (node_update, f)


# speedup vs baseline: 8.3819x; 2.9529x over previous
import functools

import jax
import jax.numpy as jnp
from jax import lax
from jax.experimental import pallas as pl
from jax.experimental.pallas import tpu as pltpu
from jax.experimental.pallas import tpu_sc as plsc

N, K, C = 10000, 32, 128
_EPS = 1e-5
_CP = C // 2


def _mlp_body(x_ref, w1t_ref, b1_ref, g1_ref, bt1_ref, w2t_ref, b2_ref,
              g2_ref, bt2_ref, pa_ref, pb_ref, f_ref, fp_ref):
    x = x_ref[...]

    h = jnp.dot(x, w1t_ref[...], preferred_element_type=jnp.float32)
    h = h + b1_ref[...]
    h = jnp.where(h > 0, h, 0.2 * h)
    mu = jnp.mean(h, axis=0, keepdims=True)
    var = jnp.mean(jnp.square(h - mu), axis=0, keepdims=True)
    h = (h - mu) * (g1_ref[...] * lax.rsqrt(var + _EPS)) + bt1_ref[...]

    h = jnp.dot(h, w2t_ref[...], preferred_element_type=jnp.float32)
    h = h + b2_ref[...]
    h = jnp.where(h > 0, h, 0.2 * h)
    mu = jnp.mean(h, axis=0, keepdims=True)
    var = jnp.mean(jnp.square(h - mu), axis=0, keepdims=True)
    h = (h - mu) * (g2_ref[...] * lax.rsqrt(var + _EPS)) + bt2_ref[...]

    f_ref[...] = h

    a = jnp.dot(h, pa_ref[...], preferred_element_type=jnp.float32)
    b = jnp.dot(h, pb_ref[...], preferred_element_type=jnp.float32)
    ua = lax.bitcast_convert_type(
        a.astype(jnp.bfloat16).astype(jnp.float32), jnp.uint32)
    ub = lax.bitcast_convert_type(
        b.astype(jnp.bfloat16).astype(jnp.float32), jnp.uint32)
    w = lax.bitwise_or(
        lax.shift_right_logical(ua, jnp.uint32(16)),
        lax.bitwise_and(ub, jnp.uint32(0xFFFF0000)),
    )
    fp_ref[...] = lax.bitcast_convert_type(w, jnp.int32)


def _mlp(x, w1t, b1, g1, bt1, w2t, b2, g2, bt2, pa, pb):
    return pl.pallas_call(
        _mlp_body,
        out_shape=(
            jax.ShapeDtypeStruct((N, C), jnp.float32),
            jax.ShapeDtypeStruct((N, _CP), jnp.int32),
        ),
    )(x, w1t, b1, g1, bt1, w2t, b2, g2, bt2, pa, pb)


_NC = 2
_NS = 16
_NW = _NC * _NS
_G = 8
_CHUNKS = 40
_PW = _G * _CHUNKS
_CHUNKS_LAST = (N - (_NW - 1) * _PW) // _G
_W16 = _CP // 16
_KU = 4


_STRIPE = 624


def _gather_mean_body(fp_hbm, idx_hbm, out_hbm, idx_v, rows0, rows1, out_v,
                      fp_sh, sem0, sem1):
    sid = lax.axis_index("s")
    wid = sid * _NC + lax.axis_index("c")
    node_base = wid * _PW

    @pl.when(sid < _NS - 1)
    def _():
        pltpu.sync_copy(fp_hbm.at[pl.ds(sid * _STRIPE, _STRIPE)],
                        fp_sh.at[pl.ds(sid * _STRIPE, _STRIPE)])

    @pl.when(sid == _NS - 1)
    def _():
        pltpu.sync_copy(fp_hbm.at[pl.ds(sid * _STRIPE, N - (_NS - 1) * _STRIPE)],
                        fp_sh.at[pl.ds(sid * _STRIPE, N - (_NS - 1) * _STRIPE)])

    plsc.subcore_barrier()

    rows = (rows0, rows1)
    sems = (sem0, sem1)

    def gather_src(ci):
        return fp_sh.at[idx_v.at[pl.ds(ci * (_G * K), _G * K)]]

    def fire(ci, b):
        pltpu.async_copy(gather_src(ci), rows[b], sems[b])

    zero = jnp.zeros((16,), jnp.float32)
    himask = jnp.full((16,), -65536, jnp.int32)

    def run(nchunks):
        pltpu.sync_copy(
            idx_hbm.at[pl.ds(node_base * K, nchunks * _G * K)],
            idx_v.at[pl.ds(0, nchunks * _G * K)])

        fire(0, 0)
        fire(1, 1)

        def pair(c0, carry):
            for b in range(2):
                ci = c0 * 2 + b
                pltpu.make_async_copy(gather_src(0), rows[b], sems[b]).wait()
                rv = rows[b]
                for g in range(_G):
                    row0 = g * K

                    def kbody(kk, accs, rv=rv, row0=row0):
                        base = row0 + kk * _KU
                        for u in range(_KU):
                            new = list(accs)
                            for v in range(_W16):
                                w = rv[base + u, pl.ds(v * 16, 16)]
                                lo = lax.bitcast_convert_type(
                                    lax.shift_left(w, 16), jnp.float32)
                                hi = lax.bitcast_convert_type(
                                    lax.bitwise_and(w, himask), jnp.float32)
                                new[2 * v] = new[2 * v] + lo
                                new[2 * v + 1] = new[2 * v + 1] + hi
                            accs = tuple(new)
                        return accs

                    accs = lax.fori_loop(0, K // _KU, kbody,
                                         (zero,) * (2 * _W16))
                    for c in range(2 * _W16):
                        out_v[g, pl.ds(c * 16, 16)] = accs[c] * (1.0 / K)

                pltpu.sync_copy(out_v,
                                out_hbm.at[pl.ds(node_base + ci * _G, _G)])

                @pl.when(ci + 2 < nchunks)
                def _():
                    fire(ci + 2, b)

            return carry

        lax.fori_loop(0, nchunks // 2, pair, 0)

    @pl.when(wid != _NW - 1)
    def _():
        run(_CHUNKS)

    @pl.when(wid == _NW - 1)
    def _():
        run(_CHUNKS_LAST)


@functools.partial(jax.jit, donate_argnums=())
def _gather_mean(fp, idx_flat):
    mesh = plsc.VectorSubcoreMesh(core_axis_name="c", subcore_axis_name="s")
    return pl.kernel(
        _gather_mean_body,
        out_type=jax.ShapeDtypeStruct((N, C), jnp.float32),
        mesh=mesh,
        scratch_types=[
            pltpu.VMEM((_PW * K,), jnp.int32),
            pltpu.VMEM((_G * K, _CP), jnp.int32),
            pltpu.VMEM((_G * K, _CP), jnp.int32),
            pltpu.VMEM((_G, C), jnp.float32),
            pltpu.VMEM_SHARED((N, _CP), jnp.int32),
            pltpu.SemaphoreType.DMA,
            pltpu.SemaphoreType.DMA,
        ],
        compiler_params=pltpu.CompilerParams(use_tc_tiling_on_sc=False),
    )(fp, idx_flat)


_COLS_A = [32 * (p // 16) + p % 16 for p in range(_CP)]
_COLS_B = [32 * (p // 16) + 16 + p % 16 for p in range(_CP)]


def kernel(input_features, node_neigh_index, prob_retained, W1, b1, g1, bt1,
           W2, b2, g2, bt2):
    del prob_retained
    pa = jax.nn.one_hot(jnp.array(_COLS_A), C, axis=0, dtype=jnp.float32)
    pb = jax.nn.one_hot(jnp.array(_COLS_B), C, axis=0, dtype=jnp.float32)
    f, fp = _mlp(
        input_features,
        W1.T,
        b1.reshape(1, C), g1.reshape(1, C), bt1.reshape(1, C),
        W2.T,
        b2.reshape(1, C), g2.reshape(1, C), bt2.reshape(1, C),
        pa, pb,
    )
    idx_flat = node_neigh_index.astype(jnp.int32).reshape(-1)
    node_update = _gather_mean(fp, idx_flat)
    return (node_update, f)


# speedup vs baseline: 9.3055x; 1.1102x over previous
import functools

import jax
import jax.numpy as jnp
from jax import lax
from jax.experimental import pallas as pl
from jax.experimental.pallas import tpu as pltpu
from jax.experimental.pallas import tpu_sc as plsc

N, K, C = 10000, 32, 128
_EPS = 1e-5
_CP = C // 2


_DN = (((1,), (1,)), ((), ()))


def _mlp_body(x_ref, w1_ref, b1_ref, g1_ref, bt1_ref, w2_ref, b2_ref,
              g2_ref, bt2_ref, pa_ref, pb_ref, f_ref, fp_ref):
    x = x_ref[...]

    h = lax.dot_general(x, w1_ref[...], _DN,
                        preferred_element_type=jnp.float32)
    h = h + b1_ref[...]
    h = jnp.where(h > 0, h, 0.2 * h)
    mu = jnp.mean(h, axis=0, keepdims=True)
    var = jnp.mean(jnp.square(h - mu), axis=0, keepdims=True)
    h = (h - mu) * (g1_ref[...] * lax.rsqrt(var + _EPS)) + bt1_ref[...]

    h = lax.dot_general(h, w2_ref[...], _DN,
                        preferred_element_type=jnp.float32)
    h = h + b2_ref[...]
    h = jnp.where(h > 0, h, 0.2 * h)
    mu = jnp.mean(h, axis=0, keepdims=True)
    var = jnp.mean(jnp.square(h - mu), axis=0, keepdims=True)
    h = (h - mu) * (g2_ref[...] * lax.rsqrt(var + _EPS)) + bt2_ref[...]

    f_ref[...] = h

    a = jnp.dot(h, pa_ref[...], preferred_element_type=jnp.float32)
    b = jnp.dot(h, pb_ref[...], preferred_element_type=jnp.float32)
    ua = lax.bitcast_convert_type(
        a.astype(jnp.bfloat16).astype(jnp.float32), jnp.int32)
    g = lax.shift_right_logical(ua, 16)
    ub = lax.bitcast_convert_type(b, jnp.int32)
    s = lax.bitwise_and(ub, jnp.int32(-2147483648))
    m = lax.bitwise_and(ub, jnp.int32(0x7FFFFFFF))
    t = lax.shift_right_logical(
        jnp.clip(m - g + jnp.int32(0x8000), 0, jnp.int32(0x7FFFFFFF)), 16)
    fp_ref[...] = lax.bitwise_or(g, lax.bitwise_or(s, lax.shift_left(t, 16)))


def _mlp(x, w1, b1, g1, bt1, w2, b2, g2, bt2, pa, pb):
    return pl.pallas_call(
        _mlp_body,
        out_shape=(
            jax.ShapeDtypeStruct((N, C), jnp.float32),
            jax.ShapeDtypeStruct((N, _CP), jnp.int32),
        ),
    )(x, w1, b1, g1, bt1, w2, b2, g2, bt2, pa, pb)


_NC = 2
_NS = 16
_NW = _NC * _NS
_G = 8
_CHUNKS = 40
_PW = _G * _CHUNKS
_CHUNKS_LAST = (N - (_NW - 1) * _PW) // _G
_W16 = _CP // 16
_KU = 8


_STRIPE = 624


def _gather_mean_body(fp_hbm, idx_hbm, out_hbm, idx_v, rows0, rows1, out_v,
                      fp_sh, sem0, sem1):
    sid = lax.axis_index("s")
    wid = sid * _NC + lax.axis_index("c")
    node_base = wid * _PW

    @pl.when(sid < _NS - 1)
    def _():
        pltpu.sync_copy(fp_hbm.at[pl.ds(sid * _STRIPE, _STRIPE)],
                        fp_sh.at[pl.ds(sid * _STRIPE, _STRIPE)])

    @pl.when(sid == _NS - 1)
    def _():
        pltpu.sync_copy(fp_hbm.at[pl.ds(sid * _STRIPE, N - (_NS - 1) * _STRIPE)],
                        fp_sh.at[pl.ds(sid * _STRIPE, N - (_NS - 1) * _STRIPE)])

    rows = (rows0, rows1)
    sems = (sem0, sem1)

    def gather_src(ci):
        return fp_sh.at[idx_v.at[pl.ds(ci * (_G * K), _G * K)]]

    def fire(ci, b):
        pltpu.async_copy(gather_src(ci), rows[b], sems[b])

    zero = jnp.zeros((16,), jnp.float32)

    def stage_idx(nchunks):
        pltpu.sync_copy(
            idx_hbm.at[pl.ds(node_base * K, nchunks * _G * K)],
            idx_v.at[pl.ds(0, nchunks * _G * K)])

    @pl.when(wid != _NW - 1)
    def _():
        stage_idx(_CHUNKS)

    @pl.when(wid == _NW - 1)
    def _():
        stage_idx(_CHUNKS_LAST)

    plsc.subcore_barrier()

    def run(nchunks):
        fire(0, 0)
        fire(1, 1)

        def pair(c0, carry):
            for b in range(2):
                ci = c0 * 2 + b
                pltpu.make_async_copy(gather_src(0), rows[b], sems[b]).wait()
                rv = rows[b]
                for g in range(_G):
                    row0 = g * K

                    def kbody(kk, accs, rv=rv, row0=row0):
                        base = row0 + kk * _KU
                        for u in range(_KU):
                            new = list(accs)
                            for v in range(_W16):
                                w = rv[base + u, pl.ds(v * 16, 16)]
                                lo = lax.bitcast_convert_type(
                                    lax.shift_left(w, 16), jnp.float32)
                                hi = lax.bitcast_convert_type(w, jnp.float32)
                                new[2 * v] = new[2 * v] + lo
                                new[2 * v + 1] = new[2 * v + 1] + hi
                            accs = tuple(new)
                        return accs

                    accs = lax.fori_loop(0, K // _KU, kbody,
                                         (zero,) * (2 * _W16))
                    for c in range(2 * _W16):
                        out_v[g, pl.ds(c * 16, 16)] = accs[c] * (1.0 / K)

                pltpu.sync_copy(out_v,
                                out_hbm.at[pl.ds(node_base + ci * _G, _G)])

                @pl.when(ci + 2 < nchunks)
                def _():
                    fire(ci + 2, b)

            return carry

        lax.fori_loop(0, nchunks // 2, pair, 0)

    @pl.when(wid != _NW - 1)
    def _():
        run(_CHUNKS)

    @pl.when(wid == _NW - 1)
    def _():
        run(_CHUNKS_LAST)


@functools.partial(jax.jit, donate_argnums=())
def _gather_mean(fp, idx_flat):
    mesh = plsc.VectorSubcoreMesh(core_axis_name="c", subcore_axis_name="s")
    return pl.kernel(
        _gather_mean_body,
        out_type=jax.ShapeDtypeStruct((N, C), jnp.float32),
        mesh=mesh,
        scratch_types=[
            pltpu.VMEM((_PW * K,), jnp.int32),
            pltpu.VMEM((_G * K, _CP), jnp.int32),
            pltpu.VMEM((_G * K, _CP), jnp.int32),
            pltpu.VMEM((_G, C), jnp.float32),
            pltpu.VMEM_SHARED((N, _CP), jnp.int32),
            pltpu.SemaphoreType.DMA,
            pltpu.SemaphoreType.DMA,
        ],
        compiler_params=pltpu.CompilerParams(use_tc_tiling_on_sc=False),
    )(fp, idx_flat)


_COLS_A = [32 * (p // 16) + p % 16 for p in range(_CP)]
_COLS_B = [32 * (p // 16) + 16 + p % 16 for p in range(_CP)]


def kernel(input_features, node_neigh_index, prob_retained, W1, b1, g1, bt1,
           W2, b2, g2, bt2):
    del prob_retained
    pa = jax.nn.one_hot(jnp.array(_COLS_A), C, axis=0, dtype=jnp.float32)
    pb = jax.nn.one_hot(jnp.array(_COLS_B), C, axis=0, dtype=jnp.float32)
    f, fp = _mlp(
        input_features,
        W1,
        b1.reshape(1, C), g1.reshape(1, C), bt1.reshape(1, C),
        W2,
        b2.reshape(1, C), g2.reshape(1, C), bt2.reshape(1, C),
        pa, pb,
    )
    idx_flat = node_neigh_index.astype(jnp.int32).reshape(-1)
    node_update = _gather_mean(fp, idx_flat)
    return (node_update, f)


# speedup vs baseline: 9.6421x; 1.0362x over previous
import functools

import jax
import jax.numpy as jnp
from jax import lax
from jax.experimental import pallas as pl
from jax.experimental.pallas import tpu as pltpu
from jax.experimental.pallas import tpu_sc as plsc

N, K, C = 10000, 32, 128
_EPS = 1e-5
_CP = C // 2


_DN = (((1,), (1,)), ((), ()))


def _mlp_body(x_ref, w1_ref, b1_ref, g1_ref, bt1_ref, w2_ref, b2_ref,
              g2_ref, bt2_ref, pa_ref, pb_ref, f_ref, fp_ref):
    x = x_ref[...]

    h = lax.dot_general(x, w1_ref[...], _DN,
                        preferred_element_type=jnp.float32)
    h = h + b1_ref[...]
    l1 = jnp.where(h > 0, h, 0.2 * h)
    mu = jnp.mean(l1, axis=0, keepdims=True)
    var = jnp.mean(jnp.square(l1), axis=0, keepdims=True) - jnp.square(mu)
    s1 = g1_ref[...] * lax.rsqrt(var + _EPS)
    t1 = bt1_ref[...] - mu * s1
    w2s = w2_ref[...] * s1
    b2f = b2_ref[...] + lax.dot_general(t1, w2_ref[...], _DN,
                                        preferred_element_type=jnp.float32)
    h = lax.dot_general(l1, w2s, _DN, preferred_element_type=jnp.float32)
    h = h + b2f
    h = jnp.where(h > 0, h, 0.2 * h)
    mu = jnp.mean(h, axis=0, keepdims=True)
    var = jnp.mean(jnp.square(h), axis=0, keepdims=True) - jnp.square(mu)
    h = (h - mu) * (g2_ref[...] * lax.rsqrt(var + _EPS)) + bt2_ref[...]

    f_ref[...] = h

    a = jnp.dot(h, pa_ref[...], preferred_element_type=jnp.float32)
    b = jnp.dot(h, pb_ref[...], preferred_element_type=jnp.float32)
    ua = lax.bitcast_convert_type(
        a.astype(jnp.bfloat16).astype(jnp.float32), jnp.int32)
    g = lax.shift_right_logical(ua, 16)
    ub = lax.bitcast_convert_type(b, jnp.int32)
    s = lax.bitwise_and(ub, jnp.int32(-2147483648))
    m = lax.bitwise_and(ub, jnp.int32(0x7FFFFFFF))
    t = lax.shift_right_logical(
        jnp.clip(m - g + jnp.int32(0x8000), 0, jnp.int32(0x7FFFFFFF)), 16)
    fp_ref[...] = lax.bitwise_or(g, lax.bitwise_or(s, lax.shift_left(t, 16)))


def _mlp(x, w1, b1, g1, bt1, w2, b2, g2, bt2, pa, pb):
    return pl.pallas_call(
        _mlp_body,
        out_shape=(
            jax.ShapeDtypeStruct((N, C), jnp.float32),
            jax.ShapeDtypeStruct((N, _CP), jnp.int32),
        ),
    )(x, w1, b1, g1, bt1, w2, b2, g2, bt2, pa, pb)


_NC = 2
_NS = 16
_NW = _NC * _NS
_G = 8
_CHUNKS = 40
_PW = _G * _CHUNKS
_CHUNKS_LAST = (N - (_NW - 1) * _PW) // _G
_W16 = _CP // 16
_KU = 8


_STRIPE = 624


def _gather_mean_body(fp_hbm, idx_hbm, out_hbm, idx_v, rows0, rows1, out_v,
                      fp_sh, sem0, sem1):
    sid = lax.axis_index("s")
    wid = sid * _NC + lax.axis_index("c")
    node_base = wid * _PW

    @pl.when(sid < _NS - 1)
    def _():
        pltpu.sync_copy(fp_hbm.at[pl.ds(sid * _STRIPE, _STRIPE)],
                        fp_sh.at[pl.ds(sid * _STRIPE, _STRIPE)])

    @pl.when(sid == _NS - 1)
    def _():
        pltpu.sync_copy(fp_hbm.at[pl.ds(sid * _STRIPE, N - (_NS - 1) * _STRIPE)],
                        fp_sh.at[pl.ds(sid * _STRIPE, N - (_NS - 1) * _STRIPE)])

    rows = (rows0, rows1)
    sems = (sem0, sem1)

    def gather_src(ci):
        return fp_sh.at[idx_v.at[pl.ds(ci * (_G * K), _G * K)]]

    def fire(ci, b):
        pltpu.async_copy(gather_src(ci), rows[b], sems[b])

    zero = jnp.zeros((16,), jnp.float32)

    def stage_idx(nchunks):
        pltpu.sync_copy(
            idx_hbm.at[pl.ds(node_base * K, nchunks * _G * K)],
            idx_v.at[pl.ds(0, nchunks * _G * K)])

    @pl.when(wid != _NW - 1)
    def _():
        stage_idx(_CHUNKS)

    @pl.when(wid == _NW - 1)
    def _():
        stage_idx(_CHUNKS_LAST)

    plsc.subcore_barrier()

    def run(nchunks):
        fire(0, 0)
        fire(1, 1)

        def pair(c0, carry):
            for b in range(2):
                ci = c0 * 2 + b
                pltpu.make_async_copy(gather_src(0), rows[b], sems[b]).wait()
                rv = rows[b]
                for g in range(_G):
                    row0 = g * K

                    def kbody(kk, accs, rv=rv, row0=row0):
                        base = row0 + kk * _KU
                        for u in range(_KU):
                            new = list(accs)
                            for v in range(_W16):
                                w = rv[base + u, pl.ds(v * 16, 16)]
                                lo = lax.bitcast_convert_type(
                                    lax.shift_left(w, 16), jnp.float32)
                                hi = lax.bitcast_convert_type(w, jnp.float32)
                                new[2 * v] = new[2 * v] + lo
                                new[2 * v + 1] = new[2 * v + 1] + hi
                            accs = tuple(new)
                        return accs

                    accs = lax.fori_loop(0, K // _KU, kbody,
                                         (zero,) * (2 * _W16))
                    for c in range(2 * _W16):
                        out_v[g, pl.ds(c * 16, 16)] = accs[c] * (1.0 / K)

                pltpu.sync_copy(out_v,
                                out_hbm.at[pl.ds(node_base + ci * _G, _G)])

                @pl.when(ci + 2 < nchunks)
                def _():
                    fire(ci + 2, b)

            return carry

        lax.fori_loop(0, nchunks // 2, pair, 0)

    @pl.when(wid != _NW - 1)
    def _():
        run(_CHUNKS)

    @pl.when(wid == _NW - 1)
    def _():
        run(_CHUNKS_LAST)


@functools.partial(jax.jit, donate_argnums=())
def _gather_mean(fp, idx_flat):
    mesh = plsc.VectorSubcoreMesh(core_axis_name="c", subcore_axis_name="s")
    return pl.kernel(
        _gather_mean_body,
        out_type=jax.ShapeDtypeStruct((N, C), jnp.float32),
        mesh=mesh,
        scratch_types=[
            pltpu.VMEM((_PW * K,), jnp.int32),
            pltpu.VMEM((_G * K, _CP), jnp.int32),
            pltpu.VMEM((_G * K, _CP), jnp.int32),
            pltpu.VMEM((_G, C), jnp.float32),
            pltpu.VMEM_SHARED((N, _CP), jnp.int32),
            pltpu.SemaphoreType.DMA,
            pltpu.SemaphoreType.DMA,
        ],
        compiler_params=pltpu.CompilerParams(use_tc_tiling_on_sc=False),
    )(fp, idx_flat)


_COLS_A = [32 * (p // 16) + p % 16 for p in range(_CP)]
_COLS_B = [32 * (p // 16) + 16 + p % 16 for p in range(_CP)]


def kernel(input_features, node_neigh_index, prob_retained, W1, b1, g1, bt1,
           W2, b2, g2, bt2):
    del prob_retained
    pa = jax.nn.one_hot(jnp.array(_COLS_A), C, axis=0, dtype=jnp.float32)
    pb = jax.nn.one_hot(jnp.array(_COLS_B), C, axis=0, dtype=jnp.float32)
    f, fp = _mlp(
        input_features,
        W1,
        b1.reshape(1, C), g1.reshape(1, C), bt1.reshape(1, C),
        W2,
        b2.reshape(1, C), g2.reshape(1, C), bt2.reshape(1, C),
        pa, pb,
    )
    idx_flat = node_neigh_index.astype(jnp.int32).reshape(-1)
    node_update = _gather_mean(fp, idx_flat)
    return (node_update, f)
